# baseline (speedup 1.0000x reference)
import functools

import jax
import jax.numpy as jnp
from jax import lax
from jax.experimental import pallas as pl
from jax.experimental.pallas import tpu as pltpu
from jax.experimental.pallas import tpu_sc as plsc

EPS = 1e-6
NC = 2
NS = 16
NW = NC * NS
L = 16
K = 128


def _sc_mesh():
    return plsc.VectorSubcoreMesh(
        core_axis_name="c", subcore_axis_name="s", num_cores=NC, num_subcores=NS
    )


def _make_degrees(E, N):
    assert E % NW == 0 and N % L == 0
    epw = E // NW
    assert epw % L == 0 and (epw * 4) % 8 == 0

    @functools.partial(
        pl.kernel,
        out_type=jax.ShapeDtypeStruct((NW * 2 * N,), jnp.float32),
        mesh=_sc_mesh(),
        compiler_params=pltpu.CompilerParams(needs_layout_passes=False),
        scratch_types=[
            pltpu.VMEM((N,), jnp.float32),
            pltpu.VMEM((N,), jnp.float32),
            pltpu.VMEM((epw,), jnp.int32),
            pltpu.VMEM((epw,), jnp.int32),
        ],
    )
    def deg_kernel(src_hbm, dst_hbm, out_hbm, hs, hd, si, di):
        c = lax.axis_index("c")
        s = lax.axis_index("s")
        wid = c * NS + s
        base = wid * epw
        zeros16 = jnp.zeros((L,), jnp.float32)
        ones16 = jnp.ones((L,), jnp.float32)

        pltpu.sync_copy(src_hbm.at[pl.ds(base, epw)], si)
        pltpu.sync_copy(dst_hbm.at[pl.ds(base, epw)], di)

        def zero_body(i, carry):
            hs[pl.ds(i * L, L)] = zeros16
            hd[pl.ds(i * L, L)] = zeros16
            return carry

        lax.fori_loop(0, N // L, zero_body, 0)

        def hist_body(i, carry):
            plsc.addupdate_scatter(hs, [si[pl.ds(i * L, L)]], ones16)
            plsc.addupdate_scatter(hd, [di[pl.ds(i * L, L)]], ones16)
            return carry

        lax.fori_loop(0, epw // L, hist_body, 0)

        pltpu.sync_copy(hs, out_hbm.at[pl.ds(wid * 2 * N, N)])
        pltpu.sync_copy(hd, out_hbm.at[pl.ds(wid * 2 * N + N, N)])

    return deg_kernel


def _make_aggregate(E_pad, N, D):
    assert E_pad % (NW * K) == 0
    cpt = E_pad // (NW * K)
    assert cpt % 8 == 0 and cpt >= 4
    NA = N + 8
    rpt = NA // NS // 8 * 8
    ztail = NA - NS * rpt
    etail = N - NS * rpt

    @functools.partial(
        pl.kernel,
        out_type=jax.ShapeDtypeStruct((NC, N, D), jnp.float32),
        mesh=_sc_mesh(),
        compiler_params=pltpu.CompilerParams(needs_layout_passes=False),
        scratch_types=[
            pltpu.VMEM_SHARED((NA, D), jnp.float32),
            pltpu.VMEM((cpt, K), jnp.int32),
            pltpu.VMEM((K,), jnp.int32),
            pltpu.VMEM((K,), jnp.int32),
            pltpu.VMEM((K, D), jnp.float32),
            pltpu.VMEM((K, D), jnp.float32),
            pltpu.SemaphoreType.DMA,
            pltpu.SemaphoreType.DMA,
            pltpu.SemaphoreType.DMA,
            pltpu.SemaphoreType.DMA,
            pltpu.SemaphoreType.DMA,
            pltpu.SemaphoreType.DMA,
        ],
    )
    def agg_kernel(h_hbm, src_hbm, dst2_hbm, zeros_hbm, out_hbm,
                   acc, di, ia, ib, rows_a, rows_b,
                   sem_a, sem_b, sem_ia, sem_ib, sem_sa, sem_sb):
        c = lax.axis_index("c")
        s = lax.axis_index("s")
        wid = c * NS + s
        ebase = wid * cpt * K

        pltpu.sync_copy(zeros_hbm.at[pl.ds(s * rpt, rpt)],
                        acc.at[pl.ds(s * rpt, rpt)])
        if ztail:
            @pl.when(s == NS - 1)
            def _():
                pltpu.sync_copy(zeros_hbm.at[pl.ds(NS * rpt, ztail)],
                                acc.at[pl.ds(NS * rpt, ztail)])

        pltpu.sync_copy(dst2_hbm.at[pl.ds(wid * cpt, cpt)], di)
        plsc.subcore_barrier()

        def idx_copy(j, buf, sem):
            pltpu.async_copy(src_hbm.at[pl.ds(ebase + j * K, K)], buf, sem)

        def idx_wait(buf, sem):
            pltpu.make_async_copy(src_hbm.at[pl.ds(ebase, K)], buf, sem).wait()

        def gather(buf_idx, buf, sem):
            pltpu.async_copy(h_hbm.at[buf_idx], buf, sem)

        def gather_wait(buf_idx, buf, sem):
            pltpu.make_async_copy(h_hbm.at[buf_idx], buf, sem).wait()

        def scatter(j, buf, sem):
            pltpu.async_copy(buf, acc.at[di.at[j]], sem, add=True)

        def scatter_wait(buf, sem):
            pltpu.make_async_copy(buf, acc.at[di.at[0]], sem).wait()

        idx_copy(0, ia, sem_ia)
        idx_copy(1, ib, sem_ib)
        idx_wait(ia, sem_ia)
        gather(ia, rows_a, sem_a)
        idx_wait(ib, sem_ib)
        gather(ib, rows_b, sem_b)

        def pipe_body(i, carry):
            gather_wait(ia, rows_a, sem_a)
            idx_copy(2 * i + 2, ia, sem_ia)
            scatter(2 * i, rows_a, sem_sa)
            gather_wait(ib, rows_b, sem_b)
            idx_copy(2 * i + 3, ib, sem_ib)
            scatter(2 * i + 1, rows_b, sem_sb)
            idx_wait(ia, sem_ia)
            scatter_wait(rows_a, sem_sa)
            gather(ia, rows_a, sem_a)
            idx_wait(ib, sem_ib)
            scatter_wait(rows_b, sem_sb)
            gather(ib, rows_b, sem_b)
            return carry

        lax.fori_loop(0, cpt // 2 - 1, pipe_body, 0)

        gather_wait(ia, rows_a, sem_a)
        scatter(cpt - 2, rows_a, sem_sa)
        gather_wait(ib, rows_b, sem_b)
        scatter(cpt - 1, rows_b, sem_sb)
        scatter_wait(rows_a, sem_sa)
        scatter_wait(rows_b, sem_sb)

        plsc.subcore_barrier()
        pltpu.sync_copy(acc.at[pl.ds(s * rpt, rpt)],
                        out_hbm.at[c, pl.ds(s * rpt, rpt)])
        if etail:
            @pl.when(s == NS - 1)
            def _():
                pltpu.sync_copy(acc.at[pl.ds(NS * rpt, etail)],
                                out_hbm.at[c, pl.ds(NS * rpt, etail)])

    return agg_kernel


def _prep(x, hist_t, a2, b2, block_n):
    N, D = x.shape

    def body(x_ref, hist_ref, a2_ref, b2_ref, h_ref):
        xb = x_ref[...]
        mean = jnp.mean(xb, axis=1, keepdims=True)
        xc = xb - mean
        std = jnp.sqrt(jnp.sum(xc * xc, axis=1, keepdims=True) / (D - 1))
        hn = a2_ref[...] * xc / (std + EPS) + b2_ref[...]
        out_deg = jnp.maximum(jnp.sum(hist_ref[...][0], axis=1), 1.0)
        h_ref[...] = hn * lax.rsqrt(out_deg)[:, None]

    return pl.pallas_call(
        body,
        grid=(N // block_n,),
        in_specs=[
            pl.BlockSpec((block_n, D), lambda i: (i, 0)),
            pl.BlockSpec((2, block_n, NW), lambda i: (0, i, 0)),
            pl.BlockSpec((1, D), lambda i: (0, 0)),
            pl.BlockSpec((1, D), lambda i: (0, 0)),
        ],
        out_specs=pl.BlockSpec((block_n, D), lambda i: (i, 0)),
        out_shape=jax.ShapeDtypeStruct((N, D), jnp.float32),
    )(x, hist_t, a2.reshape(1, D), b2.reshape(1, D))


def _finish(parts, hist_t, x, W, b, block_n):
    N, D = x.shape

    def body(parts_ref, hist_ref, x_ref, w_ref, b_ref, out_ref):
        agg = parts_ref[0] + parts_ref[1]
        in_deg = jnp.maximum(jnp.sum(hist_ref[...][1], axis=1), 1.0)
        agg = agg * lax.rsqrt(in_deg)[:, None]
        out = jnp.dot(agg, w_ref[...], preferred_element_type=jnp.float32)
        out_ref[...] = jnp.maximum(out + b_ref[...], 0.0) + x_ref[...]

    return pl.pallas_call(
        body,
        grid=(N // block_n,),
        in_specs=[
            pl.BlockSpec((NC, block_n, D), lambda i: (0, i, 0)),
            pl.BlockSpec((2, block_n, NW), lambda i: (0, i, 0)),
            pl.BlockSpec((block_n, D), lambda i: (i, 0)),
            pl.BlockSpec((D, D), lambda i: (0, 0)),
            pl.BlockSpec((1, D), lambda i: (0, 0)),
        ],
        out_specs=pl.BlockSpec((block_n, D), lambda i: (i, 0)),
        out_shape=jax.ShapeDtypeStruct((N, D), jnp.float32),
    )(parts, hist_t, x, W, b.reshape(1, D))


def kernel(x, edge_index, W, b, a2, b2):
    N, D = x.shape
    E = edge_index.shape[1]
    src = edge_index[0]
    dst = edge_index[1]

    hist = _make_degrees(E, N)(src, dst).reshape(NW, 2, N)
    hist_t = jnp.transpose(hist, (1, 2, 0))

    block_n = 1000 if N % 1000 == 0 else 8
    h = _prep(x, hist_t, a2, b2, block_n)

    epw_pad = -(-(E // NW) // (8 * K)) * (8 * K)
    E_pad = epw_pad * NW
    pad = E_pad - E
    src_p = jnp.concatenate([src, jnp.zeros((pad,), jnp.int32)])
    dst_p = jnp.concatenate([dst, jnp.full((pad,), N, jnp.int32)])
    dst2 = dst_p.reshape(E_pad // K, K)
    zeros = jnp.zeros((N + 8, D), jnp.float32)
    parts = _make_aggregate(E_pad, N, D)(h, src_p, dst2, zeros)

    return _finish(parts, hist_t, x, W, b, block_n)


# speedup vs baseline: 1.0020x; 1.0020x over previous
import functools

import jax
import jax.numpy as jnp
from jax import lax
from jax.experimental import pallas as pl
from jax.experimental.pallas import tpu as pltpu
from jax.experimental.pallas import tpu_sc as plsc

EPS = 1e-6
NC = 2
NS = 16
NW = NC * NS
L = 16
K = 128


def _sc_mesh():
    return plsc.VectorSubcoreMesh(
        core_axis_name="c", subcore_axis_name="s", num_cores=NC, num_subcores=NS
    )


def _make_degrees(E, N):
    assert E % NW == 0 and N % L == 0
    epw = E // NW
    assert epw % L == 0 and (epw * 4) % 8 == 0

    @functools.partial(
        pl.kernel,
        out_type=jax.ShapeDtypeStruct((NW * 2 * N,), jnp.float32),
        mesh=_sc_mesh(),
        compiler_params=pltpu.CompilerParams(needs_layout_passes=False),
        scratch_types=[
            pltpu.VMEM((N,), jnp.float32),
            pltpu.VMEM((N,), jnp.float32),
            pltpu.VMEM((epw,), jnp.int32),
            pltpu.VMEM((epw,), jnp.int32),
        ],
    )
    def deg_kernel(src_hbm, dst_hbm, out_hbm, hs, hd, si, di):
        c = lax.axis_index("c")
        s = lax.axis_index("s")
        wid = c * NS + s
        base = wid * epw
        zeros16 = jnp.zeros((L,), jnp.float32)
        ones16 = jnp.ones((L,), jnp.float32)

        pltpu.sync_copy(src_hbm.at[pl.ds(base, epw)], si)
        pltpu.sync_copy(dst_hbm.at[pl.ds(base, epw)], di)

        def zero_body(i, carry):
            hs[pl.ds(i * L, L)] = zeros16
            hd[pl.ds(i * L, L)] = zeros16
            return carry

        lax.fori_loop(0, N // L, zero_body, 0)

        def hist_body(i, carry):
            plsc.addupdate_scatter(hs, [si[pl.ds(i * L, L)]], ones16)
            plsc.addupdate_scatter(hd, [di[pl.ds(i * L, L)]], ones16)
            return carry

        lax.fori_loop(0, epw // L, hist_body, 0)

        pltpu.sync_copy(hs, out_hbm.at[pl.ds(wid * 2 * N, N)])
        pltpu.sync_copy(hd, out_hbm.at[pl.ds(wid * 2 * N + N, N)])

    return deg_kernel


def _make_aggregate(E_pad, N, D):
    assert E_pad % (NW * K) == 0
    cpt = E_pad // (NW * K)
    assert cpt % 8 == 0 and cpt >= 4
    NA = N + 8
    rpt = NA // NS // 8 * 8
    ztail = NA - NS * rpt
    etail = N - NS * rpt

    @functools.partial(
        pl.kernel,
        out_type=jax.ShapeDtypeStruct((NC, N, D), jnp.float32),
        mesh=_sc_mesh(),
        compiler_params=pltpu.CompilerParams(needs_layout_passes=False),
        scratch_types=[
            pltpu.VMEM_SHARED((NA, D), jnp.float32),
            pltpu.VMEM((K,), jnp.int32),
            pltpu.VMEM((K,), jnp.int32),
            pltpu.VMEM((K,), jnp.int32),
            pltpu.VMEM((K,), jnp.int32),
            pltpu.VMEM((K, D), jnp.float32),
            pltpu.VMEM((K, D), jnp.float32),
            pltpu.SemaphoreType.DMA,
            pltpu.SemaphoreType.DMA,
            pltpu.SemaphoreType.DMA,
            pltpu.SemaphoreType.DMA,
            pltpu.SemaphoreType.DMA,
            pltpu.SemaphoreType.DMA,
            pltpu.SemaphoreType.DMA,
            pltpu.SemaphoreType.DMA,
        ],
    )
    def agg_kernel(h_hbm, src_hbm, dst_hbm, zeros_hbm, out_hbm,
                   acc, ia, ib, ja, jb, rows_a, rows_b,
                   sem_a, sem_b, sem_ia, sem_ib, sem_ja, sem_jb,
                   sem_sa, sem_sb):
        c = lax.axis_index("c")
        s = lax.axis_index("s")
        wid = c * NS + s
        ebase = wid * cpt * K

        pltpu.sync_copy(zeros_hbm.at[pl.ds(s * rpt, rpt)],
                        acc.at[pl.ds(s * rpt, rpt)])
        if ztail:
            @pl.when(s == NS - 1)
            def _():
                pltpu.sync_copy(zeros_hbm.at[pl.ds(NS * rpt, ztail)],
                                acc.at[pl.ds(NS * rpt, ztail)])
        plsc.subcore_barrier()

        def idx_copy(hbm, j, buf, sem):
            pltpu.async_copy(hbm.at[pl.ds(ebase + j * K, K)], buf, sem)

        def idx_wait(hbm, buf, sem):
            pltpu.make_async_copy(hbm.at[pl.ds(ebase, K)], buf, sem).wait()

        def gather(buf_idx, buf, sem):
            pltpu.async_copy(h_hbm.at[buf_idx], buf, sem)

        def gather_wait(buf_idx, buf, sem):
            pltpu.make_async_copy(h_hbm.at[buf_idx], buf, sem).wait()

        def scatter(buf, buf_idx, sem):
            pltpu.async_copy(buf, acc.at[buf_idx], sem, add=True)

        def scatter_wait(buf, buf_idx, sem):
            pltpu.make_async_copy(buf, acc.at[buf_idx], sem).wait()

        idx_copy(src_hbm, 0, ia, sem_ia)
        idx_copy(src_hbm, 1, ib, sem_ib)
        idx_copy(dst_hbm, 0, ja, sem_ja)
        idx_copy(dst_hbm, 1, jb, sem_jb)
        idx_wait(src_hbm, ia, sem_ia)
        gather(ia, rows_a, sem_a)
        idx_wait(src_hbm, ib, sem_ib)
        gather(ib, rows_b, sem_b)

        def pipe_body(i, carry):
            gather_wait(ia, rows_a, sem_a)
            idx_copy(src_hbm, 2 * i + 2, ia, sem_ia)
            idx_wait(dst_hbm, ja, sem_ja)
            scatter(rows_a, ja, sem_sa)
            gather_wait(ib, rows_b, sem_b)
            idx_copy(src_hbm, 2 * i + 3, ib, sem_ib)
            idx_wait(dst_hbm, jb, sem_jb)
            scatter(rows_b, jb, sem_sb)
            idx_wait(src_hbm, ia, sem_ia)
            scatter_wait(rows_a, ja, sem_sa)
            gather(ia, rows_a, sem_a)
            idx_copy(dst_hbm, 2 * i + 2, ja, sem_ja)
            idx_wait(src_hbm, ib, sem_ib)
            scatter_wait(rows_b, jb, sem_sb)
            gather(ib, rows_b, sem_b)
            idx_copy(dst_hbm, 2 * i + 3, jb, sem_jb)
            return carry

        lax.fori_loop(0, cpt // 2 - 1, pipe_body, 0)

        gather_wait(ia, rows_a, sem_a)
        idx_wait(dst_hbm, ja, sem_ja)
        scatter(rows_a, ja, sem_sa)
        gather_wait(ib, rows_b, sem_b)
        idx_wait(dst_hbm, jb, sem_jb)
        scatter(rows_b, jb, sem_sb)
        scatter_wait(rows_a, ja, sem_sa)
        scatter_wait(rows_b, jb, sem_sb)

        plsc.subcore_barrier()
        pltpu.sync_copy(acc.at[pl.ds(s * rpt, rpt)],
                        out_hbm.at[c, pl.ds(s * rpt, rpt)])
        if etail:
            @pl.when(s == NS - 1)
            def _():
                pltpu.sync_copy(acc.at[pl.ds(NS * rpt, etail)],
                                out_hbm.at[c, pl.ds(NS * rpt, etail)])

    return agg_kernel


def _prep(x, hist_t, a2, b2, block_n):
    N, D = x.shape

    def body(x_ref, hist_ref, a2_ref, b2_ref, h_ref):
        xb = x_ref[...]
        mean = jnp.mean(xb, axis=1, keepdims=True)
        xc = xb - mean
        std = jnp.sqrt(jnp.sum(xc * xc, axis=1, keepdims=True) / (D - 1))
        hn = a2_ref[...] * xc / (std + EPS) + b2_ref[...]
        out_deg = jnp.maximum(jnp.sum(hist_ref[...][0], axis=1), 1.0)
        h_ref[...] = hn * lax.rsqrt(out_deg)[:, None]

    return pl.pallas_call(
        body,
        grid=(N // block_n,),
        in_specs=[
            pl.BlockSpec((block_n, D), lambda i: (i, 0)),
            pl.BlockSpec((2, block_n, NW), lambda i: (0, i, 0)),
            pl.BlockSpec((1, D), lambda i: (0, 0)),
            pl.BlockSpec((1, D), lambda i: (0, 0)),
        ],
        out_specs=pl.BlockSpec((block_n, D), lambda i: (i, 0)),
        out_shape=jax.ShapeDtypeStruct((N, D), jnp.float32),
    )(x, hist_t, a2.reshape(1, D), b2.reshape(1, D))


def _finish(parts, hist_t, x, W, b, block_n):
    N, D = x.shape

    def body(parts_ref, hist_ref, x_ref, w_ref, b_ref, out_ref):
        agg = parts_ref[0] + parts_ref[1]
        in_deg = jnp.maximum(jnp.sum(hist_ref[...][1], axis=1), 1.0)
        agg = agg * lax.rsqrt(in_deg)[:, None]
        out = jnp.dot(agg, w_ref[...], preferred_element_type=jnp.float32)
        out_ref[...] = jnp.maximum(out + b_ref[...], 0.0) + x_ref[...]

    return pl.pallas_call(
        body,
        grid=(N // block_n,),
        in_specs=[
            pl.BlockSpec((NC, block_n, D), lambda i: (0, i, 0)),
            pl.BlockSpec((2, block_n, NW), lambda i: (0, i, 0)),
            pl.BlockSpec((block_n, D), lambda i: (i, 0)),
            pl.BlockSpec((D, D), lambda i: (0, 0)),
            pl.BlockSpec((1, D), lambda i: (0, 0)),
        ],
        out_specs=pl.BlockSpec((block_n, D), lambda i: (i, 0)),
        out_shape=jax.ShapeDtypeStruct((N, D), jnp.float32),
    )(parts, hist_t, x, W, b.reshape(1, D))


def kernel(x, edge_index, W, b, a2, b2):
    N, D = x.shape
    E = edge_index.shape[1]
    src = edge_index[0]
    dst = edge_index[1]

    hist = _make_degrees(E, N)(src, dst).reshape(NW, 2, N)
    hist_t = jnp.transpose(hist, (1, 2, 0))

    block_n = 1000 if N % 1000 == 0 else 8
    h = _prep(x, hist_t, a2, b2, block_n)

    epw_pad = -(-(E // NW) // (8 * K)) * (8 * K)
    E_pad = epw_pad * NW
    pad = E_pad - E
    src_p = jnp.concatenate([src, jnp.zeros((pad,), jnp.int32)])
    dst_p = jnp.concatenate([dst, jnp.full((pad,), N, jnp.int32)])
    zeros = jnp.zeros((N + 8, D), jnp.float32)
    parts = _make_aggregate(E_pad, N, D)(h, src_p, dst_p, zeros)

    return _finish(parts, hist_t, x, W, b, block_n)


# speedup vs baseline: 2.1284x; 2.1242x over previous
import functools

import jax
import jax.numpy as jnp
from jax import lax
from jax.experimental import pallas as pl
from jax.experimental.pallas import tpu as pltpu
from jax.experimental.pallas import tpu_sc as plsc

EPS = 1e-6
NC = 2
NS = 16
NW = NC * NS
L = 16
K = 128


def _sc_mesh():
    return plsc.VectorSubcoreMesh(
        core_axis_name="c", subcore_axis_name="s", num_cores=NC, num_subcores=NS
    )


def _make_degrees(E, N):
    assert E % NW == 0 and N % L == 0
    epw = E // NW
    assert epw % L == 0 and (epw * 4) % 8 == 0

    @functools.partial(
        pl.kernel,
        out_type=jax.ShapeDtypeStruct((NW * 2 * N,), jnp.float32),
        mesh=_sc_mesh(),
        compiler_params=pltpu.CompilerParams(needs_layout_passes=False),
        scratch_types=[
            pltpu.VMEM((N,), jnp.float32),
            pltpu.VMEM((N,), jnp.float32),
            pltpu.VMEM((epw,), jnp.int32),
            pltpu.VMEM((epw,), jnp.int32),
        ],
    )
    def deg_kernel(src_hbm, dst_hbm, out_hbm, hs, hd, si, di):
        c = lax.axis_index("c")
        s = lax.axis_index("s")
        wid = c * NS + s
        base = wid * epw
        zeros16 = jnp.zeros((L,), jnp.float32)
        ones16 = jnp.ones((L,), jnp.float32)

        pltpu.sync_copy(src_hbm.at[pl.ds(base, epw)], si)
        pltpu.sync_copy(dst_hbm.at[pl.ds(base, epw)], di)

        def zero_body(i, carry):
            hs[pl.ds(i * L, L)] = zeros16
            hd[pl.ds(i * L, L)] = zeros16
            return carry

        lax.fori_loop(0, N // L, zero_body, 0)

        def hist_body(i, carry):
            plsc.addupdate_scatter(hs, [si[pl.ds(i * L, L)]], ones16)
            plsc.addupdate_scatter(hd, [di[pl.ds(i * L, L)]], ones16)
            return carry

        lax.fori_loop(0, epw // L, hist_body, 0)

        pltpu.sync_copy(hs, out_hbm.at[pl.ds(wid * 2 * N, N)])
        pltpu.sync_copy(hd, out_hbm.at[pl.ds(wid * 2 * N + N, N)])

    return deg_kernel


def _make_aggregate(E_pad, N, D):
    assert E_pad % (NW * K) == 0
    cpt = E_pad // (NW * K)
    assert cpt % 2 == 0 and cpt >= 4
    NA = N
    rpt = NA // NS // 8 * 8
    ztail = NA - NS * rpt
    etail = N - NS * rpt

    @functools.partial(
        pl.kernel,
        out_type=jax.ShapeDtypeStruct((NC, N, D), jnp.float32),
        mesh=_sc_mesh(),
        compiler_params=pltpu.CompilerParams(needs_layout_passes=False),
        scratch_types=[
            pltpu.VMEM_SHARED((NA, D), jnp.float32),
            pltpu.VMEM((K,), jnp.int32),
            pltpu.VMEM((K,), jnp.int32),
            pltpu.VMEM((K,), jnp.int32),
            pltpu.VMEM((K,), jnp.int32),
            pltpu.VMEM((K, D), jnp.float32),
            pltpu.VMEM((K, D), jnp.float32),
            pltpu.SemaphoreType.DMA,
            pltpu.SemaphoreType.DMA,
            pltpu.SemaphoreType.DMA,
            pltpu.SemaphoreType.DMA,
            pltpu.SemaphoreType.DMA,
            pltpu.SemaphoreType.DMA,
            pltpu.SemaphoreType.DMA,
            pltpu.SemaphoreType.DMA,
        ],
    )
    def agg_kernel(h_hbm, src_hbm, dst_hbm, zeros_hbm, out_hbm,
                   acc, ia, ib, ja, jb, rows_a, rows_b,
                   sem_a, sem_b, sem_ia, sem_ib, sem_ja, sem_jb,
                   sem_sa, sem_sb):
        c = lax.axis_index("c")
        s = lax.axis_index("s")
        wid = c * NS + s
        ebase = wid * cpt * K

        pltpu.sync_copy(zeros_hbm.at[pl.ds(s * rpt, rpt)],
                        acc.at[pl.ds(s * rpt, rpt)])
        if ztail:
            @pl.when(s == NS - 1)
            def _():
                pltpu.sync_copy(zeros_hbm.at[pl.ds(NS * rpt, ztail)],
                                acc.at[pl.ds(NS * rpt, ztail)])
        plsc.subcore_barrier()

        def idx_copy(hbm, j, buf, sem):
            pltpu.async_copy(hbm.at[pl.ds(ebase + j * K, K)], buf, sem)

        def idx_wait(hbm, buf, sem):
            pltpu.make_async_copy(hbm.at[pl.ds(ebase, K)], buf, sem).wait()

        def gather(buf_idx, buf, sem):
            pltpu.async_copy(h_hbm.at[buf_idx], buf, sem)

        def gather_wait(buf_idx, buf, sem):
            pltpu.make_async_copy(h_hbm.at[buf_idx], buf, sem).wait()

        def scatter(buf, buf_idx, sem):
            pltpu.async_copy(buf, acc.at[buf_idx], sem, add=True)

        def scatter_wait(buf, buf_idx, sem):
            pltpu.make_async_copy(buf, acc.at[buf_idx], sem).wait()

        idx_copy(src_hbm, 0, ia, sem_ia)
        idx_copy(src_hbm, 1, ib, sem_ib)
        idx_copy(dst_hbm, 0, ja, sem_ja)
        idx_copy(dst_hbm, 1, jb, sem_jb)
        idx_wait(src_hbm, ia, sem_ia)
        gather(ia, rows_a, sem_a)
        idx_wait(src_hbm, ib, sem_ib)
        gather(ib, rows_b, sem_b)

        def pipe_body(i, carry):
            gather_wait(ia, rows_a, sem_a)
            idx_copy(src_hbm, 2 * i + 2, ia, sem_ia)
            idx_wait(dst_hbm, ja, sem_ja)
            scatter(rows_a, ja, sem_sa)
            gather_wait(ib, rows_b, sem_b)
            idx_copy(src_hbm, 2 * i + 3, ib, sem_ib)
            idx_wait(dst_hbm, jb, sem_jb)
            scatter(rows_b, jb, sem_sb)
            idx_wait(src_hbm, ia, sem_ia)
            scatter_wait(rows_a, ja, sem_sa)
            gather(ia, rows_a, sem_a)
            idx_copy(dst_hbm, 2 * i + 2, ja, sem_ja)
            idx_wait(src_hbm, ib, sem_ib)
            scatter_wait(rows_b, jb, sem_sb)
            gather(ib, rows_b, sem_b)
            idx_copy(dst_hbm, 2 * i + 3, jb, sem_jb)
            return carry

        lax.fori_loop(0, cpt // 2 - 1, pipe_body, 0)

        gather_wait(ia, rows_a, sem_a)
        idx_wait(dst_hbm, ja, sem_ja)
        scatter(rows_a, ja, sem_sa)
        gather_wait(ib, rows_b, sem_b)
        idx_wait(dst_hbm, jb, sem_jb)
        scatter(rows_b, jb, sem_sb)
        scatter_wait(rows_a, ja, sem_sa)
        scatter_wait(rows_b, jb, sem_sb)

        plsc.subcore_barrier()
        pltpu.sync_copy(acc.at[pl.ds(s * rpt, rpt)],
                        out_hbm.at[c, pl.ds(s * rpt, rpt)])
        if etail:
            @pl.when(s == NS - 1)
            def _():
                pltpu.sync_copy(acc.at[pl.ds(NS * rpt, etail)],
                                out_hbm.at[c, pl.ds(NS * rpt, etail)])

    return agg_kernel


def _prep(x, hist_t, a2, b2, block_n):
    N, D = x.shape

    def body(x_ref, hist_ref, a2_ref, b2_ref, h_ref):
        xb = x_ref[...]
        mean = jnp.mean(xb, axis=1, keepdims=True)
        xc = xb - mean
        std = jnp.sqrt(jnp.sum(xc * xc, axis=1, keepdims=True) / (D - 1))
        hn = a2_ref[...] * xc / (std + EPS) + b2_ref[...]
        out_deg = jnp.maximum(jnp.sum(hist_ref[...][0], axis=1), 1.0)
        h_ref[...] = hn * lax.rsqrt(out_deg)[:, None]

    return pl.pallas_call(
        body,
        grid=(N // block_n,),
        in_specs=[
            pl.BlockSpec((block_n, D), lambda i: (i, 0)),
            pl.BlockSpec((2, block_n, NW), lambda i: (0, i, 0)),
            pl.BlockSpec((1, D), lambda i: (0, 0)),
            pl.BlockSpec((1, D), lambda i: (0, 0)),
        ],
        out_specs=pl.BlockSpec((block_n, D), lambda i: (i, 0)),
        out_shape=jax.ShapeDtypeStruct((N, D), jnp.float32),
    )(x, hist_t, a2.reshape(1, D), b2.reshape(1, D))


def _finish(parts, hist_t, x, W, b, block_n):
    N, D = x.shape

    def body(parts_ref, hist_ref, x_ref, w_ref, b_ref, out_ref):
        agg = parts_ref[0] + parts_ref[1]
        in_deg = jnp.maximum(jnp.sum(hist_ref[...][1], axis=1), 1.0)
        agg = agg * lax.rsqrt(in_deg)[:, None]
        out = jnp.dot(agg, w_ref[...], preferred_element_type=jnp.float32)
        out_ref[...] = jnp.maximum(out + b_ref[...], 0.0) + x_ref[...]

    return pl.pallas_call(
        body,
        grid=(N // block_n,),
        in_specs=[
            pl.BlockSpec((NC, block_n, D), lambda i: (0, i, 0)),
            pl.BlockSpec((2, block_n, NW), lambda i: (0, i, 0)),
            pl.BlockSpec((block_n, D), lambda i: (i, 0)),
            pl.BlockSpec((D, D), lambda i: (0, 0)),
            pl.BlockSpec((1, D), lambda i: (0, 0)),
        ],
        out_specs=pl.BlockSpec((block_n, D), lambda i: (i, 0)),
        out_shape=jax.ShapeDtypeStruct((N, D), jnp.float32),
    )(parts, hist_t, x, W, b.reshape(1, D))


def kernel(x, edge_index, W, b, a2, b2):
    N, D = x.shape
    E = edge_index.shape[1]
    src = edge_index[0]
    dst = edge_index[1]

    hist = _make_degrees(E, N)(src, dst).reshape(NW, 2, N)
    hist_t = jnp.transpose(hist, (1, 2, 0))

    block_n = 1000 if N % 1000 == 0 else 8
    h = _prep(x, hist_t, a2, b2, block_n)

    epw = E // NW
    epw_pad = -(-epw // (2 * K)) * (2 * K)
    E_pad = epw_pad * NW
    padw = epw_pad - epw
    pad_src = (N + jnp.arange(NW * padw, dtype=jnp.int32) % 8).reshape(NW, padw)
    pad_dst = (jnp.arange(NW * padw, dtype=jnp.int32) % N).reshape(NW, padw)
    src_p = jnp.concatenate([src.reshape(NW, epw), pad_src], axis=1).reshape(-1)
    dst_p = jnp.concatenate([dst.reshape(NW, epw), pad_dst], axis=1).reshape(-1)
    h_pad = jnp.concatenate([h, jnp.zeros((8, D), jnp.float32)], axis=0)
    zeros = jnp.zeros((N, D), jnp.float32)
    parts = _make_aggregate(E_pad, N, D)(h_pad, src_p, dst_p, zeros)

    return _finish(parts, hist_t, x, W, b, block_n)


# speedup vs baseline: 2.4254x; 1.1396x over previous
import functools

import jax
import jax.numpy as jnp
from jax import lax
from jax.experimental import pallas as pl
from jax.experimental.pallas import tpu as pltpu
from jax.experimental.pallas import tpu_sc as plsc

EPS = 1e-6
NC = 2
NS = 16
NW = NC * NS
L = 16
K = 128


def _sc_mesh():
    return plsc.VectorSubcoreMesh(
        core_axis_name="c", subcore_axis_name="s", num_cores=NC, num_subcores=NS
    )


def _make_degrees(E, N):
    assert E % NW == 0 and N % L == 0
    epw = E // NW
    assert epw % L == 0 and (epw * 4) % 8 == 0

    @functools.partial(
        pl.kernel,
        out_type=jax.ShapeDtypeStruct((NW * 2 * N,), jnp.float32),
        mesh=_sc_mesh(),
        compiler_params=pltpu.CompilerParams(needs_layout_passes=False),
        scratch_types=[
            pltpu.VMEM((N,), jnp.float32),
            pltpu.VMEM((N,), jnp.float32),
            pltpu.VMEM((epw,), jnp.int32),
            pltpu.VMEM((epw,), jnp.int32),
        ],
    )
    def deg_kernel(src_hbm, dst_hbm, out_hbm, hs, hd, si, di):
        c = lax.axis_index("c")
        s = lax.axis_index("s")
        wid = c * NS + s
        base = wid * epw
        zeros16 = jnp.zeros((L,), jnp.float32)
        ones16 = jnp.ones((L,), jnp.float32)

        pltpu.sync_copy(src_hbm.at[pl.ds(base, epw)], si)
        pltpu.sync_copy(dst_hbm.at[pl.ds(base, epw)], di)

        def zero_body(i, carry):
            hs[pl.ds(i * L, L)] = zeros16
            hd[pl.ds(i * L, L)] = zeros16
            return carry

        lax.fori_loop(0, N // L, zero_body, 0)

        def hist_body(i, carry):
            plsc.addupdate_scatter(hs, [si[pl.ds(i * L, L)]], ones16)
            plsc.addupdate_scatter(hd, [di[pl.ds(i * L, L)]], ones16)
            return carry

        lax.fori_loop(0, epw // L, hist_body, 0)

        pltpu.sync_copy(hs, out_hbm.at[pl.ds(wid * 2 * N, N)])
        pltpu.sync_copy(hd, out_hbm.at[pl.ds(wid * 2 * N + N, N)])

    return deg_kernel


def _make_aggregate(E_pad, N, D):
    assert E_pad % (NW * K) == 0
    cpt = E_pad // (NW * K)
    assert cpt % 2 == 0 and cpt >= 4
    NA = N + 256
    rpt = NA // NS // 8 * 8
    ztail = NA - NS * rpt
    rpe = N // NS // 8 * 8
    etail = N - NS * rpe

    @functools.partial(
        pl.kernel,
        out_type=jax.ShapeDtypeStruct((NC, N, D), jnp.float32),
        mesh=_sc_mesh(),
        compiler_params=pltpu.CompilerParams(needs_layout_passes=False),
        scratch_types=[
            pltpu.VMEM_SHARED((NA, D), jnp.float32),
            pltpu.VMEM((K,), jnp.int32),
            pltpu.VMEM((K,), jnp.int32),
            pltpu.VMEM((K,), jnp.int32),
            pltpu.VMEM((K,), jnp.int32),
            pltpu.VMEM((K, D), jnp.float32),
            pltpu.VMEM((K, D), jnp.float32),
            pltpu.SemaphoreType.DMA,
            pltpu.SemaphoreType.DMA,
            pltpu.SemaphoreType.DMA,
            pltpu.SemaphoreType.DMA,
            pltpu.SemaphoreType.DMA,
            pltpu.SemaphoreType.DMA,
            pltpu.SemaphoreType.DMA,
            pltpu.SemaphoreType.DMA,
        ],
    )
    def agg_kernel(h_hbm, src_hbm, dst_hbm, zeros_hbm, out_hbm,
                   acc, ia, ib, ja, jb, rows_a, rows_b,
                   sem_a, sem_b, sem_ia, sem_ib, sem_ja, sem_jb,
                   sem_sa, sem_sb):
        c = lax.axis_index("c")
        s = lax.axis_index("s")
        wid = c * NS + s
        ebase = wid * cpt * K

        pltpu.sync_copy(zeros_hbm.at[pl.ds(s * rpt, rpt)],
                        acc.at[pl.ds(s * rpt, rpt)])
        if ztail:
            @pl.when(s == NS - 1)
            def _():
                pltpu.sync_copy(zeros_hbm.at[pl.ds(NS * rpt, ztail)],
                                acc.at[pl.ds(NS * rpt, ztail)])
        plsc.subcore_barrier()

        def idx_copy(hbm, j, buf, sem):
            pltpu.async_copy(hbm.at[pl.ds(ebase + j * K, K)], buf, sem)

        def idx_wait(hbm, buf, sem):
            pltpu.make_async_copy(hbm.at[pl.ds(ebase, K)], buf, sem).wait()

        def gather(buf_idx, buf, sem):
            pltpu.async_copy(h_hbm.at[buf_idx], buf, sem)

        def gather_wait(buf_idx, buf, sem):
            pltpu.make_async_copy(h_hbm.at[buf_idx], buf, sem).wait()

        def scatter(buf, buf_idx, sem):
            pltpu.async_copy(buf, acc.at[buf_idx], sem, add=True)

        def scatter_wait(buf, buf_idx, sem):
            pltpu.make_async_copy(buf, acc.at[buf_idx], sem).wait()

        idx_copy(src_hbm, 0, ia, sem_ia)
        idx_copy(src_hbm, 1, ib, sem_ib)
        idx_copy(dst_hbm, 0, ja, sem_ja)
        idx_copy(dst_hbm, 1, jb, sem_jb)
        idx_wait(src_hbm, ia, sem_ia)
        gather(ia, rows_a, sem_a)
        idx_wait(src_hbm, ib, sem_ib)
        gather(ib, rows_b, sem_b)

        def pipe_body(i, carry):
            gather_wait(ia, rows_a, sem_a)
            idx_copy(src_hbm, 2 * i + 2, ia, sem_ia)
            idx_wait(dst_hbm, ja, sem_ja)
            scatter(rows_a, ja, sem_sa)
            gather_wait(ib, rows_b, sem_b)
            idx_copy(src_hbm, 2 * i + 3, ib, sem_ib)
            idx_wait(dst_hbm, jb, sem_jb)
            scatter(rows_b, jb, sem_sb)
            idx_wait(src_hbm, ia, sem_ia)
            scatter_wait(rows_a, ja, sem_sa)
            gather(ia, rows_a, sem_a)
            idx_copy(dst_hbm, 2 * i + 2, ja, sem_ja)
            idx_wait(src_hbm, ib, sem_ib)
            scatter_wait(rows_b, jb, sem_sb)
            gather(ib, rows_b, sem_b)
            idx_copy(dst_hbm, 2 * i + 3, jb, sem_jb)
            return carry

        lax.fori_loop(0, cpt // 2 - 1, pipe_body, 0)

        gather_wait(ia, rows_a, sem_a)
        idx_wait(dst_hbm, ja, sem_ja)
        scatter(rows_a, ja, sem_sa)
        gather_wait(ib, rows_b, sem_b)
        idx_wait(dst_hbm, jb, sem_jb)
        scatter(rows_b, jb, sem_sb)
        scatter_wait(rows_a, ja, sem_sa)
        scatter_wait(rows_b, jb, sem_sb)

        plsc.subcore_barrier()
        pltpu.sync_copy(acc.at[pl.ds(s * rpe, rpe)],
                        out_hbm.at[c, pl.ds(s * rpe, rpe)])
        if etail:
            @pl.when(s == NS - 1)
            def _():
                pltpu.sync_copy(acc.at[pl.ds(NS * rpe, etail)],
                                out_hbm.at[c, pl.ds(NS * rpe, etail)])

    return agg_kernel


def _prep(x, hist_t, a2, b2, block_n):
    N, D = x.shape

    def body(x_ref, hist_ref, a2_ref, b2_ref, h_ref):
        xb = x_ref[...]
        mean = jnp.mean(xb, axis=1, keepdims=True)
        xc = xb - mean
        std = jnp.sqrt(jnp.sum(xc * xc, axis=1, keepdims=True) / (D - 1))
        hn = a2_ref[...] * xc / (std + EPS) + b2_ref[...]
        out_deg = jnp.maximum(jnp.sum(hist_ref[...][0], axis=1), 1.0)
        h_ref[...] = hn * lax.rsqrt(out_deg)[:, None]

    return pl.pallas_call(
        body,
        grid=(N // block_n,),
        in_specs=[
            pl.BlockSpec((block_n, D), lambda i: (i, 0)),
            pl.BlockSpec((2, block_n, NW), lambda i: (0, i, 0)),
            pl.BlockSpec((1, D), lambda i: (0, 0)),
            pl.BlockSpec((1, D), lambda i: (0, 0)),
        ],
        out_specs=pl.BlockSpec((block_n, D), lambda i: (i, 0)),
        out_shape=jax.ShapeDtypeStruct((N, D), jnp.float32),
    )(x, hist_t, a2.reshape(1, D), b2.reshape(1, D))


def _finish(parts, hist_t, x, W, b, block_n):
    N, D = x.shape

    def body(parts_ref, hist_ref, x_ref, w_ref, b_ref, out_ref):
        agg = parts_ref[0] + parts_ref[1]
        in_deg = jnp.maximum(jnp.sum(hist_ref[...][1], axis=1), 1.0)
        agg = agg * lax.rsqrt(in_deg)[:, None]
        out = jnp.dot(agg, w_ref[...], preferred_element_type=jnp.float32)
        out_ref[...] = jnp.maximum(out + b_ref[...], 0.0) + x_ref[...]

    return pl.pallas_call(
        body,
        grid=(N // block_n,),
        in_specs=[
            pl.BlockSpec((NC, block_n, D), lambda i: (0, i, 0)),
            pl.BlockSpec((2, block_n, NW), lambda i: (0, i, 0)),
            pl.BlockSpec((block_n, D), lambda i: (i, 0)),
            pl.BlockSpec((D, D), lambda i: (0, 0)),
            pl.BlockSpec((1, D), lambda i: (0, 0)),
        ],
        out_specs=pl.BlockSpec((block_n, D), lambda i: (i, 0)),
        out_shape=jax.ShapeDtypeStruct((N, D), jnp.float32),
    )(parts, hist_t, x, W, b.reshape(1, D))


def kernel(x, edge_index, W, b, a2, b2):
    N, D = x.shape
    E = edge_index.shape[1]
    src = edge_index[0]
    dst = edge_index[1]

    hist = _make_degrees(E, N)(src, dst).reshape(NW, 2, N)
    hist_t = jnp.transpose(hist, (1, 2, 0))

    block_n = 1000 if N % 1000 == 0 else 8
    h = _prep(x, hist_t, a2, b2, block_n)

    epw = E // NW
    epw_pad = -(-epw // (2 * K)) * (2 * K)
    E_pad = epw_pad * NW
    padw = epw_pad - epw
    pad_src = (jnp.arange(NW * padw, dtype=jnp.int32) % N).reshape(NW, padw)
    pad_dst = (N + jnp.arange(NW * padw, dtype=jnp.int32) % 256).reshape(NW, padw)
    src_p = jnp.concatenate([src.reshape(NW, epw), pad_src], axis=1).reshape(-1)
    dst_p = jnp.concatenate([dst.reshape(NW, epw), pad_dst], axis=1).reshape(-1)
    zeros = jnp.zeros((N + 256, D), jnp.float32)
    parts = _make_aggregate(E_pad, N, D)(h, src_p, dst_p, zeros)

    return _finish(parts, hist_t, x, W, b, block_n)


# speedup vs baseline: 2.6748x; 1.1028x over previous
import functools

import jax
import jax.numpy as jnp
from jax import lax
from jax.experimental import pallas as pl
from jax.experimental.pallas import tpu as pltpu
from jax.experimental.pallas import tpu_sc as plsc

EPS = 1e-6
NC = 2
NS = 16
NW = NC * NS
L = 16
K = 128


def _sc_mesh():
    return plsc.VectorSubcoreMesh(
        core_axis_name="c", subcore_axis_name="s", num_cores=NC, num_subcores=NS
    )


def _make_degrees(E, N):
    assert E % NW == 0 and N % L == 0
    epw = E // NW
    assert epw % L == 0 and (epw * 4) % 8 == 0

    @functools.partial(
        pl.kernel,
        out_type=jax.ShapeDtypeStruct((NW * 2 * N,), jnp.float32),
        mesh=_sc_mesh(),
        compiler_params=pltpu.CompilerParams(needs_layout_passes=False),
        scratch_types=[
            pltpu.VMEM((N,), jnp.float32),
            pltpu.VMEM((N,), jnp.float32),
            pltpu.VMEM((epw,), jnp.int32),
            pltpu.VMEM((epw,), jnp.int32),
        ],
    )
    def deg_kernel(src_hbm, dst_hbm, out_hbm, hs, hd, si, di):
        c = lax.axis_index("c")
        s = lax.axis_index("s")
        wid = c * NS + s
        base = wid * epw
        zeros16 = jnp.zeros((L,), jnp.float32)
        ones16 = jnp.ones((L,), jnp.float32)

        pltpu.sync_copy(src_hbm.at[pl.ds(base, epw)], si)
        pltpu.sync_copy(dst_hbm.at[pl.ds(base, epw)], di)

        def zero_body(i, carry):
            hs[pl.ds(i * L, L)] = zeros16
            hd[pl.ds(i * L, L)] = zeros16
            return carry

        lax.fori_loop(0, N // L, zero_body, 0)

        def hist_body(i, carry):
            plsc.addupdate_scatter(hs, [si[pl.ds(i * L, L)]], ones16)
            plsc.addupdate_scatter(hd, [di[pl.ds(i * L, L)]], ones16)
            return carry

        lax.fori_loop(0, epw // L, hist_body, 0)

        pltpu.sync_copy(hs, out_hbm.at[pl.ds(wid * 2 * N, N)])
        pltpu.sync_copy(hd, out_hbm.at[pl.ds(wid * 2 * N + N, N)])

    return deg_kernel


def _make_aggregate(E_pad, N, D, NB, NTRASH):
    assert E_pad % (NW * K) == 0
    cpt = E_pad // (NW * K)
    assert cpt % NB == 0 and cpt >= 2 * NB
    NA = N + NTRASH
    rpt = NA // NS // 8 * 8
    ztail = NA - NS * rpt
    rpe = N // NS // 8 * 8
    etail = N - NS * rpe

    @functools.partial(
        pl.kernel,
        out_type=jax.ShapeDtypeStruct((NC, N, D), jnp.float32),
        mesh=_sc_mesh(),
        compiler_params=pltpu.CompilerParams(needs_layout_passes=False),
        scratch_types=(
            [pltpu.VMEM_SHARED((NA, D), jnp.float32)]
            + [pltpu.VMEM((K,), jnp.int32) for _ in range(2 * NB)]
            + [pltpu.VMEM((K, D), jnp.float32) for _ in range(NB)]
            + [pltpu.SemaphoreType.DMA for _ in range(4 * NB)]
        ),
    )
    def agg_kernel(h_hbm, src_hbm, dst_hbm, zeros_hbm, out_hbm,
                   acc, *scratch):
        I = scratch[:NB]
        J = scratch[NB:2 * NB]
        R = scratch[2 * NB:3 * NB]
        sg = scratch[3 * NB:4 * NB]
        si_ = scratch[4 * NB:5 * NB]
        sj = scratch[5 * NB:6 * NB]
        ss = scratch[6 * NB:7 * NB]
        c = lax.axis_index("c")
        s = lax.axis_index("s")
        wid = c * NS + s
        ebase = wid * cpt * K

        pltpu.sync_copy(zeros_hbm.at[pl.ds(s * rpt, rpt)],
                        acc.at[pl.ds(s * rpt, rpt)])
        if ztail:
            @pl.when(s == NS - 1)
            def _():
                pltpu.sync_copy(zeros_hbm.at[pl.ds(NS * rpt, ztail)],
                                acc.at[pl.ds(NS * rpt, ztail)])
        plsc.subcore_barrier()

        def idx_copy(hbm, j, buf, sem):
            pltpu.async_copy(hbm.at[pl.ds(ebase + j * K, K)], buf, sem)

        def idx_wait(hbm, buf, sem):
            pltpu.make_async_copy(hbm.at[pl.ds(ebase, K)], buf, sem).wait()

        def gather(buf_idx, buf, sem):
            pltpu.async_copy(h_hbm.at[buf_idx], buf, sem)

        def gather_wait(buf_idx, buf, sem):
            pltpu.make_async_copy(h_hbm.at[buf_idx], buf, sem).wait()

        def scatter(buf, buf_idx, sem):
            pltpu.async_copy(buf, acc.at[buf_idx], sem, add=True)

        def scatter_wait(buf, buf_idx, sem):
            pltpu.make_async_copy(buf, acc.at[buf_idx], sem).wait()

        for p in range(NB):
            idx_copy(src_hbm, p, I[p], si_[p])
            idx_copy(dst_hbm, p, J[p], sj[p])
        for p in range(NB):
            idx_wait(src_hbm, I[p], si_[p])
            gather(I[p], R[p], sg[p])

        def pipe_body(i, carry):
            for p in range(NB):
                gather_wait(I[p], R[p], sg[p])
                idx_copy(src_hbm, NB * i + NB + p, I[p], si_[p])
                idx_wait(dst_hbm, J[p], sj[p])
                scatter(R[p], J[p], ss[p])
            for p in range(NB):
                idx_wait(src_hbm, I[p], si_[p])
                scatter_wait(R[p], J[p], ss[p])
                gather(I[p], R[p], sg[p])
                idx_copy(dst_hbm, NB * i + NB + p, J[p], sj[p])
            return carry

        lax.fori_loop(0, cpt // NB - 1, pipe_body, 0)

        for p in range(NB):
            gather_wait(I[p], R[p], sg[p])
            idx_wait(dst_hbm, J[p], sj[p])
            scatter(R[p], J[p], ss[p])
        for p in range(NB):
            scatter_wait(R[p], J[p], ss[p])

        plsc.subcore_barrier()
        pltpu.sync_copy(acc.at[pl.ds(s * rpe, rpe)],
                        out_hbm.at[c, pl.ds(s * rpe, rpe)])
        if etail:
            @pl.when(s == NS - 1)
            def _():
                pltpu.sync_copy(acc.at[pl.ds(NS * rpe, etail)],
                                out_hbm.at[c, pl.ds(NS * rpe, etail)])

    return agg_kernel


def _prep(x, hist_t, a2, b2, block_n):
    N, D = x.shape

    def body(x_ref, hist_ref, a2_ref, b2_ref, h_ref):
        xb = x_ref[...]
        mean = jnp.mean(xb, axis=1, keepdims=True)
        xc = xb - mean
        std = jnp.sqrt(jnp.sum(xc * xc, axis=1, keepdims=True) / (D - 1))
        hn = a2_ref[...] * xc / (std + EPS) + b2_ref[...]
        out_deg = jnp.maximum(jnp.sum(hist_ref[...][0], axis=1), 1.0)
        h_ref[...] = hn * lax.rsqrt(out_deg)[:, None]

    return pl.pallas_call(
        body,
        grid=(N // block_n,),
        in_specs=[
            pl.BlockSpec((block_n, D), lambda i: (i, 0)),
            pl.BlockSpec((2, block_n, NW), lambda i: (0, i, 0)),
            pl.BlockSpec((1, D), lambda i: (0, 0)),
            pl.BlockSpec((1, D), lambda i: (0, 0)),
        ],
        out_specs=pl.BlockSpec((block_n, D), lambda i: (i, 0)),
        out_shape=jax.ShapeDtypeStruct((N, D), jnp.float32),
    )(x, hist_t, a2.reshape(1, D), b2.reshape(1, D))


def _finish(parts, hist_t, x, W, b, block_n):
    N, D = x.shape

    def body(parts_ref, hist_ref, x_ref, w_ref, b_ref, out_ref):
        agg = parts_ref[0] + parts_ref[1]
        in_deg = jnp.maximum(jnp.sum(hist_ref[...][1], axis=1), 1.0)
        agg = agg * lax.rsqrt(in_deg)[:, None]
        out = jnp.dot(agg, w_ref[...], preferred_element_type=jnp.float32)
        out_ref[...] = jnp.maximum(out + b_ref[...], 0.0) + x_ref[...]

    return pl.pallas_call(
        body,
        grid=(N // block_n,),
        in_specs=[
            pl.BlockSpec((NC, block_n, D), lambda i: (0, i, 0)),
            pl.BlockSpec((2, block_n, NW), lambda i: (0, i, 0)),
            pl.BlockSpec((block_n, D), lambda i: (i, 0)),
            pl.BlockSpec((D, D), lambda i: (0, 0)),
            pl.BlockSpec((1, D), lambda i: (0, 0)),
        ],
        out_specs=pl.BlockSpec((block_n, D), lambda i: (i, 0)),
        out_shape=jax.ShapeDtypeStruct((N, D), jnp.float32),
    )(parts, hist_t, x, W, b.reshape(1, D))


def kernel(x, edge_index, W, b, a2, b2):
    N, D = x.shape
    E = edge_index.shape[1]
    src = edge_index[0]
    dst = edge_index[1]

    hist = _make_degrees(E, N)(src, dst).reshape(NW, 2, N)
    hist_t = jnp.transpose(hist, (1, 2, 0))

    block_n = 1000 if N % 1000 == 0 else 8
    h = _prep(x, hist_t, a2, b2, block_n)

    NB = 3
    NTRASH = 64
    epw = E // NW
    epw_pad = -(-epw // (NB * K)) * (NB * K)
    E_pad = epw_pad * NW
    padw = epw_pad - epw
    pad_src = (jnp.arange(NW * padw, dtype=jnp.int32) % N).reshape(NW, padw)
    pad_dst = (N + jnp.arange(NW * padw, dtype=jnp.int32) % NTRASH).reshape(NW, padw)
    src_p = jnp.concatenate([src.reshape(NW, epw), pad_src], axis=1).reshape(-1)
    dst_p = jnp.concatenate([dst.reshape(NW, epw), pad_dst], axis=1).reshape(-1)
    zeros = jnp.zeros((N + NTRASH, D), jnp.float32)
    parts = _make_aggregate(E_pad, N, D, NB, NTRASH)(h, src_p, dst_p, zeros)

    return _finish(parts, hist_t, x, W, b, block_n)


# speedup vs baseline: 2.7818x; 1.0400x over previous
import functools

import jax
import jax.numpy as jnp
from jax import lax
from jax.experimental import pallas as pl
from jax.experimental.pallas import tpu as pltpu
from jax.experimental.pallas import tpu_sc as plsc

EPS = 1e-6
NC = 2
NS = 16
NW = NC * NS
L = 16
K = 128


def _sc_mesh():
    return plsc.VectorSubcoreMesh(
        core_axis_name="c", subcore_axis_name="s", num_cores=NC, num_subcores=NS
    )


def _make_degrees(E, N):
    assert E % NW == 0 and N % L == 0
    epw = E // NW
    assert epw % L == 0 and (epw * 4) % 8 == 0

    @functools.partial(
        pl.kernel,
        out_type=jax.ShapeDtypeStruct((NW * 2 * N,), jnp.float32),
        mesh=_sc_mesh(),
        compiler_params=pltpu.CompilerParams(needs_layout_passes=False),
        scratch_types=[
            pltpu.VMEM((N,), jnp.float32),
            pltpu.VMEM((N,), jnp.float32),
            pltpu.VMEM((epw,), jnp.int32),
            pltpu.VMEM((epw,), jnp.int32),
        ],
    )
    def deg_kernel(src_hbm, dst_hbm, out_hbm, hs, hd, si, di):
        c = lax.axis_index("c")
        s = lax.axis_index("s")
        wid = c * NS + s
        base = wid * epw
        zeros16 = jnp.zeros((L,), jnp.float32)
        ones16 = jnp.ones((L,), jnp.float32)

        pltpu.sync_copy(src_hbm.at[pl.ds(base, epw)], si)
        pltpu.sync_copy(dst_hbm.at[pl.ds(base, epw)], di)

        def zero_body(i, carry):
            hs[pl.ds(i * L, L)] = zeros16
            hd[pl.ds(i * L, L)] = zeros16
            return carry

        lax.fori_loop(0, N // L, zero_body, 0)

        def hist_body(i, carry):
            plsc.addupdate_scatter(hs, [si[pl.ds(i * L, L)]], ones16)
            plsc.addupdate_scatter(hd, [di[pl.ds(i * L, L)]], ones16)
            return carry

        lax.fori_loop(0, epw // L, hist_body, 0)

        pltpu.sync_copy(hs, out_hbm.at[pl.ds(wid * 2 * N, N)])
        pltpu.sync_copy(hd, out_hbm.at[pl.ds(wid * 2 * N + N, N)])

    return deg_kernel


def _make_aggregate(E_pad, N, D, NB, NTRASH):
    assert E_pad % (NW * K) == 0
    cpt = E_pad // (NW * K)
    assert cpt % NB == 0 and cpt >= 2 * NB
    NA = N + NTRASH
    rpt = NA // NS // 8 * 8
    ztail = NA - NS * rpt
    rpe = N // NS // 8 * 8
    etail = N - NS * rpe

    @functools.partial(
        pl.kernel,
        out_type=jax.ShapeDtypeStruct((NC, N, D), jnp.float32),
        mesh=_sc_mesh(),
        compiler_params=pltpu.CompilerParams(needs_layout_passes=False),
        scratch_types=(
            [pltpu.VMEM_SHARED((NA, D), jnp.float32)]
            + [pltpu.VMEM((K,), jnp.int32) for _ in range(2 * NB)]
            + [pltpu.VMEM((K, D), jnp.float32) for _ in range(NB)]
            + [pltpu.SemaphoreType.DMA for _ in range(4 * NB)]
        ),
    )
    def agg_kernel(h_hbm, src_hbm, dst_hbm, zeros_hbm, out_hbm,
                   acc, *scratch):
        I = scratch[:NB]
        J = scratch[NB:2 * NB]
        R = scratch[2 * NB:3 * NB]
        sg = scratch[3 * NB:4 * NB]
        si_ = scratch[4 * NB:5 * NB]
        sj = scratch[5 * NB:6 * NB]
        ss = scratch[6 * NB:7 * NB]
        c = lax.axis_index("c")
        s = lax.axis_index("s")
        wid = c * NS + s
        ebase = wid * cpt * K

        pltpu.sync_copy(zeros_hbm.at[pl.ds(s * rpt, rpt)],
                        acc.at[pl.ds(s * rpt, rpt)])
        if ztail:
            @pl.when(s == NS - 1)
            def _():
                pltpu.sync_copy(zeros_hbm.at[pl.ds(NS * rpt, ztail)],
                                acc.at[pl.ds(NS * rpt, ztail)])
        plsc.subcore_barrier()

        def idx_copy(hbm, j, buf, sem):
            pltpu.async_copy(hbm.at[pl.ds(ebase + j * K, K)], buf, sem)

        def idx_wait(hbm, buf, sem):
            pltpu.make_async_copy(hbm.at[pl.ds(ebase, K)], buf, sem).wait()

        def gather(buf_idx, buf, sem):
            pltpu.async_copy(h_hbm.at[buf_idx], buf, sem)

        def gather_wait(buf_idx, buf, sem):
            pltpu.make_async_copy(h_hbm.at[buf_idx], buf, sem).wait()

        def scatter(buf, buf_idx, sem):
            pltpu.async_copy(buf, acc.at[buf_idx], sem, add=True)

        def scatter_wait(buf, buf_idx, sem):
            pltpu.make_async_copy(buf, acc.at[buf_idx], sem).wait()

        for p in range(NB):
            idx_copy(src_hbm, p, I[p], si_[p])
            idx_copy(dst_hbm, p, J[p], sj[p])
        for p in range(NB):
            idx_wait(src_hbm, I[p], si_[p])
            gather(I[p], R[p], sg[p])

        def pipe_body(i, carry):
            for p in range(NB):
                gather_wait(I[p], R[p], sg[p])
                idx_copy(src_hbm, NB * i + NB + p, I[p], si_[p])
                idx_wait(dst_hbm, J[p], sj[p])
                scatter(R[p], J[p], ss[p])
            for p in range(NB):
                idx_wait(src_hbm, I[p], si_[p])
                scatter_wait(R[p], J[p], ss[p])
                gather(I[p], R[p], sg[p])
                idx_copy(dst_hbm, NB * i + NB + p, J[p], sj[p])
            return carry

        lax.fori_loop(0, cpt // NB - 1, pipe_body, 0)

        for p in range(NB):
            gather_wait(I[p], R[p], sg[p])
            idx_wait(dst_hbm, J[p], sj[p])
            scatter(R[p], J[p], ss[p])
        for p in range(NB):
            scatter_wait(R[p], J[p], ss[p])

        plsc.subcore_barrier()
        pltpu.sync_copy(acc.at[pl.ds(s * rpe, rpe)],
                        out_hbm.at[c, pl.ds(s * rpe, rpe)])
        if etail:
            @pl.when(s == NS - 1)
            def _():
                pltpu.sync_copy(acc.at[pl.ds(NS * rpe, etail)],
                                out_hbm.at[c, pl.ds(NS * rpe, etail)])

    return agg_kernel


def _prep(x, hist, a2, b2, block_n):
    N, D = x.shape

    def body(x_ref, hist_ref, a2_ref, b2_ref, h_ref):
        xb = x_ref[...]
        mean = jnp.mean(xb, axis=1, keepdims=True)
        xc = xb - mean
        std = jnp.sqrt(jnp.sum(xc * xc, axis=1, keepdims=True) / (D - 1))
        hn = a2_ref[...] * xc / (std + EPS) + b2_ref[...]
        out_deg = jnp.maximum(jnp.sum(hist_ref[...][:, 0, :], axis=0), 1.0)
        h_ref[...] = hn * lax.rsqrt(out_deg)[:, None]

    return pl.pallas_call(
        body,
        grid=(pl.cdiv(N, block_n),),
        in_specs=[
            pl.BlockSpec((block_n, D), lambda i: (i, 0)),
            pl.BlockSpec((NW, 2, block_n), lambda i: (0, 0, i)),
            pl.BlockSpec((1, D), lambda i: (0, 0)),
            pl.BlockSpec((1, D), lambda i: (0, 0)),
        ],
        out_specs=pl.BlockSpec((block_n, D), lambda i: (i, 0)),
        out_shape=jax.ShapeDtypeStruct((N, D), jnp.float32),
    )(x, hist, a2.reshape(1, D), b2.reshape(1, D))


def _finish(parts, hist, x, W, b, block_n):
    N, D = x.shape

    def body(parts_ref, hist_ref, x_ref, w_ref, b_ref, out_ref):
        agg = parts_ref[0] + parts_ref[1]
        in_deg = jnp.maximum(jnp.sum(hist_ref[...][:, 1, :], axis=0), 1.0)
        agg = agg * lax.rsqrt(in_deg)[:, None]
        out = jnp.dot(agg, w_ref[...], preferred_element_type=jnp.float32)
        out_ref[...] = jnp.maximum(out + b_ref[...], 0.0) + x_ref[...]

    return pl.pallas_call(
        body,
        grid=(pl.cdiv(N, block_n),),
        in_specs=[
            pl.BlockSpec((NC, block_n, D), lambda i: (0, i, 0)),
            pl.BlockSpec((NW, 2, block_n), lambda i: (0, 0, i)),
            pl.BlockSpec((block_n, D), lambda i: (i, 0)),
            pl.BlockSpec((D, D), lambda i: (0, 0)),
            pl.BlockSpec((1, D), lambda i: (0, 0)),
        ],
        out_specs=pl.BlockSpec((block_n, D), lambda i: (i, 0)),
        out_shape=jax.ShapeDtypeStruct((N, D), jnp.float32),
    )(parts, hist, x, W, b.reshape(1, D))


def kernel(x, edge_index, W, b, a2, b2):
    N, D = x.shape
    E = edge_index.shape[1]
    src = edge_index[0]
    dst = edge_index[1]

    hist = _make_degrees(E, N)(src, dst).reshape(NW, 2, N)

    block_n = 1024
    h = _prep(x, hist, a2, b2, block_n)

    NB = 3
    NTRASH = 64
    epw = E // NW
    epw_pad = -(-epw // (NB * K)) * (NB * K)
    E_pad = epw_pad * NW
    padw = epw_pad - epw
    pad_src = (jnp.arange(NW * padw, dtype=jnp.int32) % N).reshape(NW, padw)
    pad_dst = (N + jnp.arange(NW * padw, dtype=jnp.int32) % NTRASH).reshape(NW, padw)
    src_p = jnp.concatenate([src.reshape(NW, epw), pad_src], axis=1).reshape(-1)
    dst_p = jnp.concatenate([dst.reshape(NW, epw), pad_dst], axis=1).reshape(-1)
    zeros = jnp.zeros((N + NTRASH, D), jnp.float32)
    parts = _make_aggregate(E_pad, N, D, NB, NTRASH)(h, src_p, dst_p, zeros)

    return _finish(parts, hist, x, W, b, block_n)


# speedup vs baseline: 2.9419x; 1.0576x over previous
import functools

import jax
import jax.numpy as jnp
from jax import lax
from jax.experimental import pallas as pl
from jax.experimental.pallas import tpu as pltpu
from jax.experimental.pallas import tpu_sc as plsc

EPS = 1e-6
NC = 2
NS = 16
NW = NC * NS
L = 16
K = 64


def _sc_mesh():
    return plsc.VectorSubcoreMesh(
        core_axis_name="c", subcore_axis_name="s", num_cores=NC, num_subcores=NS
    )


def _make_degrees(E, N):
    assert E % NW == 0 and N % L == 0
    epw = E // NW
    assert epw % L == 0 and (epw * 4) % 8 == 0

    @functools.partial(
        pl.kernel,
        out_type=jax.ShapeDtypeStruct((NW * 2 * N,), jnp.float32),
        mesh=_sc_mesh(),
        compiler_params=pltpu.CompilerParams(needs_layout_passes=False),
        scratch_types=[
            pltpu.VMEM((N,), jnp.float32),
            pltpu.VMEM((N,), jnp.float32),
            pltpu.VMEM((epw,), jnp.int32),
            pltpu.VMEM((epw,), jnp.int32),
        ],
    )
    def deg_kernel(src_hbm, dst_hbm, out_hbm, hs, hd, si, di):
        c = lax.axis_index("c")
        s = lax.axis_index("s")
        wid = c * NS + s
        base = wid * epw
        zeros16 = jnp.zeros((L,), jnp.float32)
        ones16 = jnp.ones((L,), jnp.float32)

        pltpu.sync_copy(src_hbm.at[pl.ds(base, epw)], si)
        pltpu.sync_copy(dst_hbm.at[pl.ds(base, epw)], di)

        def zero_body(i, carry):
            hs[pl.ds(i * L, L)] = zeros16
            hd[pl.ds(i * L, L)] = zeros16
            return carry

        lax.fori_loop(0, N // L, zero_body, 0)

        def hist_body(i, carry):
            plsc.addupdate_scatter(hs, [si[pl.ds(i * L, L)]], ones16)
            plsc.addupdate_scatter(hd, [di[pl.ds(i * L, L)]], ones16)
            return carry

        lax.fori_loop(0, epw // L, hist_body, 0)

        pltpu.sync_copy(hs, out_hbm.at[pl.ds(wid * 2 * N, N)])
        pltpu.sync_copy(hd, out_hbm.at[pl.ds(wid * 2 * N + N, N)])

    return deg_kernel


def _make_aggregate(E_pad, N, D, NB, NTRASH):
    assert E_pad % (NW * K) == 0
    cpt = E_pad // (NW * K)
    assert cpt % NB == 0 and cpt >= 2 * NB
    NA = N + NTRASH
    rpt = NA // NS // 8 * 8
    ztail = NA - NS * rpt
    rpe = N // NS // 8 * 8
    etail = N - NS * rpe

    @functools.partial(
        pl.kernel,
        out_type=jax.ShapeDtypeStruct((NC, N, D), jnp.float32),
        mesh=_sc_mesh(),
        compiler_params=pltpu.CompilerParams(needs_layout_passes=False),
        scratch_types=(
            [pltpu.VMEM_SHARED((NA, D), jnp.float32)]
            + [pltpu.VMEM((K,), jnp.int32) for _ in range(2 * NB)]
            + [pltpu.VMEM((K, D), jnp.float32) for _ in range(NB)]
            + [pltpu.SemaphoreType.DMA for _ in range(4 * NB)]
        ),
    )
    def agg_kernel(h_hbm, src_hbm, dst_hbm, zeros_hbm, out_hbm,
                   acc, *scratch):
        I = scratch[:NB]
        J = scratch[NB:2 * NB]
        R = scratch[2 * NB:3 * NB]
        sg = scratch[3 * NB:4 * NB]
        si_ = scratch[4 * NB:5 * NB]
        sj = scratch[5 * NB:6 * NB]
        ss = scratch[6 * NB:7 * NB]
        c = lax.axis_index("c")
        s = lax.axis_index("s")
        wid = c * NS + s
        ebase = wid * cpt * K

        pltpu.sync_copy(zeros_hbm.at[pl.ds(s * rpt, rpt)],
                        acc.at[pl.ds(s * rpt, rpt)])
        if ztail:
            @pl.when(s == NS - 1)
            def _():
                pltpu.sync_copy(zeros_hbm.at[pl.ds(NS * rpt, ztail)],
                                acc.at[pl.ds(NS * rpt, ztail)])
        plsc.subcore_barrier()

        def idx_copy(hbm, j, buf, sem):
            pltpu.async_copy(hbm.at[pl.ds(ebase + j * K, K)], buf, sem)

        def idx_wait(hbm, buf, sem):
            pltpu.make_async_copy(hbm.at[pl.ds(ebase, K)], buf, sem).wait()

        def gather(buf_idx, buf, sem):
            pltpu.async_copy(h_hbm.at[buf_idx], buf, sem)

        def gather_wait(buf_idx, buf, sem):
            pltpu.make_async_copy(h_hbm.at[buf_idx], buf, sem).wait()

        def scatter(buf, buf_idx, sem):
            pltpu.async_copy(buf, acc.at[buf_idx], sem, add=True)

        def scatter_wait(buf, buf_idx, sem):
            pltpu.make_async_copy(buf, acc.at[buf_idx], sem).wait()

        for p in range(NB):
            idx_copy(src_hbm, p, I[p], si_[p])
            idx_copy(dst_hbm, p, J[p], sj[p])
        for p in range(NB):
            idx_wait(src_hbm, I[p], si_[p])
            gather(I[p], R[p], sg[p])

        def pipe_body(i, carry):
            for p in range(NB):
                gather_wait(I[p], R[p], sg[p])
                idx_copy(src_hbm, NB * i + NB + p, I[p], si_[p])
                idx_wait(dst_hbm, J[p], sj[p])
                scatter(R[p], J[p], ss[p])
            for p in range(NB):
                idx_wait(src_hbm, I[p], si_[p])
                scatter_wait(R[p], J[p], ss[p])
                gather(I[p], R[p], sg[p])
                idx_copy(dst_hbm, NB * i + NB + p, J[p], sj[p])
            return carry

        lax.fori_loop(0, cpt // NB - 1, pipe_body, 0)

        for p in range(NB):
            gather_wait(I[p], R[p], sg[p])
            idx_wait(dst_hbm, J[p], sj[p])
            scatter(R[p], J[p], ss[p])
        for p in range(NB):
            scatter_wait(R[p], J[p], ss[p])

        plsc.subcore_barrier()
        pltpu.sync_copy(acc.at[pl.ds(s * rpe, rpe)],
                        out_hbm.at[c, pl.ds(s * rpe, rpe)])
        if etail:
            @pl.when(s == NS - 1)
            def _():
                pltpu.sync_copy(acc.at[pl.ds(NS * rpe, etail)],
                                out_hbm.at[c, pl.ds(NS * rpe, etail)])

    return agg_kernel


def _prep(x, hist, a2, b2, block_n):
    N, D = x.shape

    def body(x_ref, hist_ref, a2_ref, b2_ref, h_ref):
        xb = x_ref[...]
        mean = jnp.mean(xb, axis=1, keepdims=True)
        xc = xb - mean
        std = jnp.sqrt(jnp.sum(xc * xc, axis=1, keepdims=True) / (D - 1))
        hn = a2_ref[...] * xc / (std + EPS) + b2_ref[...]
        out_deg = jnp.maximum(jnp.sum(hist_ref[...][:, 0, :], axis=0), 1.0)
        h_ref[...] = hn * lax.rsqrt(out_deg)[:, None]

    return pl.pallas_call(
        body,
        grid=(pl.cdiv(N, block_n),),
        in_specs=[
            pl.BlockSpec((block_n, D), lambda i: (i, 0)),
            pl.BlockSpec((NW, 2, block_n), lambda i: (0, 0, i)),
            pl.BlockSpec((1, D), lambda i: (0, 0)),
            pl.BlockSpec((1, D), lambda i: (0, 0)),
        ],
        out_specs=pl.BlockSpec((block_n, D), lambda i: (i, 0)),
        out_shape=jax.ShapeDtypeStruct((N, D), jnp.float32),
    )(x, hist, a2.reshape(1, D), b2.reshape(1, D))


def _finish(parts, hist, x, W, b, block_n):
    N, D = x.shape

    def body(parts_ref, hist_ref, x_ref, w_ref, b_ref, out_ref):
        agg = parts_ref[0] + parts_ref[1]
        in_deg = jnp.maximum(jnp.sum(hist_ref[...][:, 1, :], axis=0), 1.0)
        agg = agg * lax.rsqrt(in_deg)[:, None]
        out = jnp.dot(agg, w_ref[...], preferred_element_type=jnp.float32)
        out_ref[...] = jnp.maximum(out + b_ref[...], 0.0) + x_ref[...]

    return pl.pallas_call(
        body,
        grid=(pl.cdiv(N, block_n),),
        in_specs=[
            pl.BlockSpec((NC, block_n, D), lambda i: (0, i, 0)),
            pl.BlockSpec((NW, 2, block_n), lambda i: (0, 0, i)),
            pl.BlockSpec((block_n, D), lambda i: (i, 0)),
            pl.BlockSpec((D, D), lambda i: (0, 0)),
            pl.BlockSpec((1, D), lambda i: (0, 0)),
        ],
        out_specs=pl.BlockSpec((block_n, D), lambda i: (i, 0)),
        out_shape=jax.ShapeDtypeStruct((N, D), jnp.float32),
    )(parts, hist, x, W, b.reshape(1, D))


def kernel(x, edge_index, W, b, a2, b2):
    N, D = x.shape
    E = edge_index.shape[1]
    src = edge_index[0]
    dst = edge_index[1]

    hist = _make_degrees(E, N)(src, dst).reshape(NW, 2, N)

    block_n = 1024
    h = _prep(x, hist, a2, b2, block_n)

    NB = 5
    NTRASH = 64
    epw = E // NW
    epw_pad = -(-epw // (NB * K)) * (NB * K)
    E_pad = epw_pad * NW
    padw = epw_pad - epw
    pad_src = (jnp.arange(NW * padw, dtype=jnp.int32) % N).reshape(NW, padw)
    pad_dst = (N + jnp.arange(NW * padw, dtype=jnp.int32) % NTRASH).reshape(NW, padw)
    src_p = jnp.concatenate([src.reshape(NW, epw), pad_src], axis=1).reshape(-1)
    dst_p = jnp.concatenate([dst.reshape(NW, epw), pad_dst], axis=1).reshape(-1)
    zeros = jnp.zeros((N + NTRASH, D), jnp.float32)
    parts = _make_aggregate(E_pad, N, D, NB, NTRASH)(h, src_p, dst_p, zeros)

    return _finish(parts, hist, x, W, b, block_n)
